# baseline (speedup 1.0000x reference)
import jax
import jax.numpy as jnp
from jax.experimental import pallas as pl
from jax.experimental.pallas import tpu as pltpu

_C = 64
_D = 128


def _fused_body(h0_ref, h1_ref, h2a_ref, h2b_ref, ws0_ref, wa0_ref, ws1_ref,
                wa1_ref, out_ref):
    ws0 = ws0_ref[...]
    wa0 = wa0_ref[...]
    h1 = h1_ref[...]
    m2a = jnp.mean(h2a_ref[...].reshape(8 * _C, 16, _D), axis=1)
    m2b = jnp.mean(h2b_ref[...].reshape(8 * _C, 16, _D), axis=1)
    m2 = jnp.concatenate([m2a, m2b], axis=0)
    out1 = jnp.maximum(
        jnp.dot(h1, ws0, preferred_element_type=jnp.float32)
        + jnp.dot(m2, wa0, preferred_element_type=jnp.float32), 0.0)
    b = jnp.mean(out1.reshape(_C, 16, _D), axis=1)
    a = jnp.mean(h1.reshape(_C, 16, _D), axis=1)
    out0 = jnp.maximum(
        jnp.dot(h0_ref[...], ws0, preferred_element_type=jnp.float32)
        + jnp.dot(a, wa0, preferred_element_type=jnp.float32), 0.0)
    out_ref[...] = (
        jnp.dot(out0, ws1_ref[...], preferred_element_type=jnp.float32)
        + jnp.dot(b, wa1_ref[...], preferred_element_type=jnp.float32))


def kernel(h0, h1, h2, W_self0, W_agg0, W_self1, W_agg1):
    B = h0.shape[0]
    grid = (B // _C,)
    w_spec = pl.BlockSpec((_D, _D), lambda i: (0, 0))
    return pl.pallas_call(
        _fused_body,
        grid=grid,
        in_specs=[
            pl.BlockSpec((_C, _D), lambda i: (i, 0)),
            pl.BlockSpec((16 * _C, _D), lambda i: (i, 0)),
            pl.BlockSpec((128 * _C, _D), lambda i: (2 * i, 0)),
            pl.BlockSpec((128 * _C, _D), lambda i: (2 * i + 1, 0)),
            w_spec, w_spec, w_spec, w_spec,
        ],
        out_specs=pl.BlockSpec((_C, _D), lambda i: (i, 0)),
        out_shape=jax.ShapeDtypeStruct((B, _D), jnp.float32),
    )(h0, h1, h2, h2, W_self0, W_agg0, W_self1, W_agg1)


# speedup vs baseline: 1.0002x; 1.0002x over previous
import jax
import jax.numpy as jnp
from jax.experimental import pallas as pl
from jax.experimental.pallas import tpu as pltpu

_C = 64
_D = 128


def _fused_body(h0_ref, h1_ref, h2_ref, ws0_ref, wa0_ref, ws1_ref, wa1_ref,
                out_ref):
    ws0 = ws0_ref[...]
    wa0 = wa0_ref[...]
    h1 = h1_ref[...]
    h2 = h2_ref[...]
    m2 = jnp.mean(h2.reshape(16 * _C, 16, _D), axis=1)
    out1 = jnp.maximum(
        jnp.dot(h1, ws0, preferred_element_type=jnp.float32)
        + jnp.dot(m2, wa0, preferred_element_type=jnp.float32), 0.0)
    b = jnp.mean(out1.reshape(_C, 16, _D), axis=1)
    a = jnp.mean(h1.reshape(_C, 16, _D), axis=1)
    out0 = jnp.maximum(
        jnp.dot(h0_ref[...], ws0, preferred_element_type=jnp.float32)
        + jnp.dot(a, wa0, preferred_element_type=jnp.float32), 0.0)
    out_ref[...] = (
        jnp.dot(out0, ws1_ref[...], preferred_element_type=jnp.float32)
        + jnp.dot(b, wa1_ref[...], preferred_element_type=jnp.float32))


def kernel(h0, h1, h2, W_self0, W_agg0, W_self1, W_agg1):
    B = h0.shape[0]
    grid = (B // _C,)
    w_spec = pl.BlockSpec((_D, _D), lambda i: (0, 0))
    return pl.pallas_call(
        _fused_body,
        grid=grid,
        in_specs=[
            pl.BlockSpec((_C, _D), lambda i: (i, 0)),
            pl.BlockSpec((16 * _C, _D), lambda i: (i, 0)),
            pl.BlockSpec((256 * _C, _D), lambda i: (i, 0)),
            w_spec, w_spec, w_spec, w_spec,
        ],
        out_specs=pl.BlockSpec((_C, _D), lambda i: (i, 0)),
        out_shape=jax.ShapeDtypeStruct((B, _D), jnp.float32),
    )(h0, h1, h2, W_self0, W_agg0, W_self1, W_agg1)


# speedup vs baseline: 1.0007x; 1.0005x over previous
import jax
import jax.numpy as jnp
from jax.experimental import pallas as pl

_C = 64
_D = 128


def _fused_body(h0_ref, h1_ref, h2_ref, ws0_ref, wa0_ref, ws1_ref, wa1_ref,
                out_ref):
    ws0 = ws0_ref[...]
    wa0 = wa0_ref[...]
    h1 = h1_ref[...]
    h2 = h2_ref[...]
    m2 = jnp.mean(h2.reshape(16 * _C, 16, _D), axis=1)
    out1 = jnp.maximum(
        jnp.dot(h1, ws0, preferred_element_type=jnp.float32)
        + jnp.dot(m2, wa0, preferred_element_type=jnp.float32), 0.0)
    b = jnp.mean(out1.reshape(_C, 16, _D), axis=1)
    a = jnp.mean(h1.reshape(_C, 16, _D), axis=1)
    out0 = jnp.maximum(
        jnp.dot(h0_ref[...], ws0, preferred_element_type=jnp.float32)
        + jnp.dot(a, wa0, preferred_element_type=jnp.float32), 0.0)
    out_ref[...] = (
        jnp.dot(out0, ws1_ref[...], preferred_element_type=jnp.float32)
        + jnp.dot(b, wa1_ref[...], preferred_element_type=jnp.float32))


def kernel(h0, h1, h2, W_self0, W_agg0, W_self1, W_agg1):
    B = h0.shape[0]
    grid = (B // _C,)
    w_spec = pl.BlockSpec((_D, _D), lambda i: (0, 0))
    return pl.pallas_call(
        _fused_body,
        grid=grid,
        in_specs=[
            pl.BlockSpec((_C, _D), lambda i: (i, 0)),
            pl.BlockSpec((16 * _C, _D), lambda i: (i, 0)),
            pl.BlockSpec((256 * _C, _D), lambda i: (i, 0)),
            w_spec, w_spec, w_spec, w_spec,
        ],
        out_specs=pl.BlockSpec((_C, _D), lambda i: (i, 0)),
        out_shape=jax.ShapeDtypeStruct((B, _D), jnp.float32),
    )(h0, h1, h2, W_self0, W_agg0, W_self1, W_agg1)
